# baseline (speedup 1.0000x reference)
import functools

import jax
import jax.numpy as jnp
from jax import lax
from jax.experimental import pallas as pl
from jax.experimental.pallas import tpu as pltpu
from jax.experimental.pallas import tpu_sc as plsc

_K = 8
_TQ = 512
_G = 16


def _round_bf16(x):
    u = jax.lax.bitcast_convert_type(x, jnp.uint32)
    lsb = jax.lax.shift_right_logical(u, jnp.uint32(16)) & jnp.uint32(1)
    r = (u + jnp.uint32(0x7FFF) + lsb) & jnp.uint32(0xFFFF0000)
    return jax.lax.bitcast_convert_type(r, jnp.float32)


def _tc_body(coef_ref, bofs_ref, q_ref, s_ref, oi_ref, ow_ref):
    q = q_ref[...]
    s = s_ref[...]
    qb = _round_bf16(q)
    sb = _round_bf16(s)
    cross2 = jnp.dot(-2.0 * qb, sb,
                     preferred_element_type=jnp.float32)
    q2 = jnp.sum(q * q, axis=1, keepdims=True)
    s2 = jnp.sum(s * s, axis=0, keepdims=True)
    d2 = jnp.maximum((q2 + s2) + cross2, 1e-12)

    ns = d2.shape[1]
    lane_f = jax.lax.broadcasted_iota(jnp.int32, d2.shape, 1).astype(jnp.float32)
    big = jnp.float32(float(ns))
    work = d2
    vals = []
    idxs = []
    for _ in range(_K):
        m = jnp.min(work, axis=1, keepdims=True)
        cand = jnp.where(work == m, lane_f, big)
        i = jnp.min(cand, axis=1, keepdims=True)
        vals.append(m)
        idxs.append(i)
        work = jnp.where(cand == i, jnp.float32(jnp.inf), work)

    v8 = jnp.concatenate(vals, axis=1)
    i8 = jnp.concatenate(idxs, axis=1).astype(jnp.int32)
    coef = coef_ref[0, 0]
    w = jnp.exp(coef * v8)
    wn = w / (jnp.sum(w, axis=1, keepdims=True) + 1e-5)
    oi_ref[...] = i8 + bofs_ref[0, 0]
    ow_ref[...] = wn


def _sc_body(qpw, f_ref, i_ref, w_ref, o_ref,
             idx_all, w_all, rows0_v, rows1_v, out0_v, out1_v,
             gsem0, gsem1, osem0, osem1):
    nc = 2
    wid = lax.axis_index("s") * nc + lax.axis_index("c")
    nchunk = qpw // _G
    rows = (rows0_v, rows1_v)
    outs = (out0_v, out1_v)
    gsems = (gsem0, gsem1)
    osems = (osem0, osem1)

    base = pl.multiple_of(wid * qpw * _K, 128)
    pltpu.sync_copy(i_ref.at[pl.ds(base, qpw * _K)], idx_all)
    pltpu.sync_copy(w_ref.at[pl.ds(base, qpw * _K)], w_all)

    def gather(c, p):
        pltpu.async_copy(f_ref.at[idx_all.at[pl.ds(c * (_G * _K), _G * _K)]],
                         rows[p], gsems[p])

    def out_store(c, p):
        qbase = pl.multiple_of(wid * qpw + c * _G, _G)
        return pltpu.make_async_copy(outs[p], o_ref.at[pl.ds(qbase, _G)],
                                     osems[p])

    def compute(c, p, first):
        pltpu.make_async_copy(f_ref.at[idx_all.at[pl.ds(c * (_G * _K), _G * _K)]],
                              rows[p], gsems[p]).wait()

        @pl.when(jnp.logical_not(first))
        def _():
            out_store(c - 2, p).wait()

        for qp in range(_G // 2):
            wvec = w_all[pl.ds(c * (_G * _K) + qp * 16, 16)]
            for h in range(2):
                qq = qp * 2 + h
                accs = [jnp.zeros((16,), jnp.float32) for _ in range(4)]
                for k in range(_K):
                    wb = wvec.at[jnp.full((16,), h * 8 + k, jnp.int32)].get(
                        mode='promise_in_bounds')
                    for j in range(4):
                        row = rows[p][qq * _K + k, pl.ds(j * 16, 16)]
                        accs[j] = accs[j] + wb * row
                for j in range(4):
                    outs[p][qq, pl.ds(j * 16, 16)] = accs[j]
        out_store(c, p).start()

    gather(0, 0)
    gather(1, 1)

    def pair(pp, _):
        c0 = pp * 2
        compute(c0, 0, c0 == 0)

        @pl.when(c0 + 2 < nchunk)
        def _():
            gather(c0 + 2, 0)

        compute(c0 + 1, 1, c0 == 0)

        @pl.when(c0 + 3 < nchunk)
        def _():
            gather(c0 + 3, 1)

        return ()

    lax.fori_loop(0, nchunk // 2, pair, ())
    out_store(nchunk - 2, 0).wait()
    out_store(nchunk - 1, 1).wait()


@jax.jit
def kernel(query_coords, sensor_coords, sensor_features, sigma):
    B, Nq, _ = query_coords.shape
    Ns, F = sensor_features.shape[1], sensor_features.shape[2]
    nq_blocks = Nq // _TQ

    q_flat = query_coords.reshape(B * Nq, 3)
    s_t = sensor_coords.T
    sigma_safe = jax.nn.softplus(sigma) + 0.01
    coef = (-1.0 / (2.0 * sigma_safe * sigma_safe)).reshape(1, 1)

    feats_flat = sensor_features.reshape(B * Ns, F)
    qpw = Nq // 32

    tc = functools.partial(
        pl.pallas_call,
        _tc_body,
        grid_spec=pltpu.PrefetchScalarGridSpec(
            num_scalar_prefetch=0,
            grid=(nq_blocks,),
            in_specs=[
                pl.BlockSpec(memory_space=pltpu.SMEM),
                pl.BlockSpec(memory_space=pltpu.SMEM),
                pl.BlockSpec((_TQ, 3), lambda i: (i, 0)),
                pl.BlockSpec((3, Ns), lambda i: (0, 0)),
            ],
            out_specs=[
                pl.BlockSpec((_TQ, _K), lambda i: (i, 0)),
                pl.BlockSpec((_TQ, _K), lambda i: (i, 0)),
            ],
        ),
        out_shape=[
            jax.ShapeDtypeStruct((Nq, _K), jnp.int32),
            jax.ShapeDtypeStruct((Nq, _K), jnp.float32),
        ],
    )
    sc = functools.partial(
        pl.kernel,
        out_type=jax.ShapeDtypeStruct((Nq, F), jnp.float32),
        mesh=plsc.VectorSubcoreMesh(core_axis_name="c", subcore_axis_name="s"),
        compiler_params=pltpu.CompilerParams(use_tc_tiling_on_sc=False),
        scratch_types=[
            pltpu.VMEM((qpw * _K,), jnp.int32),
            pltpu.VMEM((qpw * _K,), jnp.float32),
            pltpu.VMEM((_G * _K, F), jnp.float32),
            pltpu.VMEM((_G * _K, F), jnp.float32),
            pltpu.VMEM((_G, F), jnp.float32),
            pltpu.VMEM((_G, F), jnp.float32),
            pltpu.SemaphoreType.DMA,
            pltpu.SemaphoreType.DMA,
            pltpu.SemaphoreType.DMA,
            pltpu.SemaphoreType.DMA,
        ],
    )(functools.partial(_sc_body, qpw))

    outs = []
    for b in range(B):
        bofs = jnp.full((1, 1), b * Ns, jnp.int32)
        idx8, w8 = tc()(coef, bofs, query_coords[b], s_t)
        outs.append(sc(feats_flat, idx8.reshape(Nq * _K), w8.reshape(Nq * _K)))
    return jnp.stack(outs)


# speedup vs baseline: 1.0072x; 1.0072x over previous
import functools

import jax
import jax.numpy as jnp
from jax import lax
from jax.experimental import pallas as pl
from jax.experimental.pallas import tpu as pltpu
from jax.experimental.pallas import tpu_sc as plsc

_K = 8
_TQ = 1024
_G = 16


def _round_bf16(x):
    u = jax.lax.bitcast_convert_type(x, jnp.uint32)
    lsb = jax.lax.shift_right_logical(u, jnp.uint32(16)) & jnp.uint32(1)
    r = (u + jnp.uint32(0x7FFF) + lsb) & jnp.uint32(0xFFFF0000)
    return jax.lax.bitcast_convert_type(r, jnp.float32)


def _tc_body(coef_ref, q_ref, s_ref, oi_ref, ow_ref):
    q = q_ref[...]
    s = s_ref[...]
    qb = _round_bf16(q)
    sb = _round_bf16(s)
    cross2 = jnp.dot(-2.0 * qb, sb,
                     preferred_element_type=jnp.float32)
    q2 = jnp.sum(q * q, axis=1, keepdims=True)
    s2 = jnp.sum(s * s, axis=0, keepdims=True)
    d2 = jnp.maximum((q2 + s2) + cross2, 1e-12)

    ns = d2.shape[1]
    lane_f = jax.lax.broadcasted_iota(jnp.int32, d2.shape, 1).astype(jnp.float32)
    big = jnp.float32(float(ns))
    work = d2
    vals = []
    idxs = []
    for _ in range(_K):
        m = jnp.min(work, axis=1, keepdims=True)
        cand = jnp.where(work == m, lane_f, big)
        i = jnp.min(cand, axis=1, keepdims=True)
        vals.append(m)
        idxs.append(i)
        work = jnp.where(cand == i, jnp.float32(jnp.inf), work)

    v8 = jnp.concatenate(vals, axis=1)
    i8 = jnp.concatenate(idxs, axis=1).astype(jnp.int32)
    coef = coef_ref[0, 0]
    w = jnp.exp(coef * v8)
    wn = w / (jnp.sum(w, axis=1, keepdims=True) + 1e-5)
    b = pl.program_id(0)
    oi_ref[...] = i8 + b * ns
    ow_ref[...] = wn


def _sc_body(f_ref, i_ref, w_ref, o_ref,
             idx_all, w_all, rows0_v, rows1_v, out0_v, out1_v,
             gsem0, gsem1, osem0, osem1):
    nc = 2
    wid = lax.axis_index("s") * nc + lax.axis_index("c")
    qpw = 512
    nchunk = qpw // _G
    rows = (rows0_v, rows1_v)
    outs = (out0_v, out1_v)
    gsems = (gsem0, gsem1)
    osems = (osem0, osem1)

    base = pl.multiple_of(wid * qpw * _K, 128)
    pltpu.sync_copy(i_ref.at[pl.ds(base, qpw * _K)], idx_all)
    pltpu.sync_copy(w_ref.at[pl.ds(base, qpw * _K)], w_all)

    def gather(c, p):
        pltpu.async_copy(f_ref.at[idx_all.at[pl.ds(c * (_G * _K), _G * _K)]],
                         rows[p], gsems[p])

    def out_store(c, p):
        qbase = pl.multiple_of(wid * qpw + c * _G, _G)
        return pltpu.make_async_copy(outs[p], o_ref.at[pl.ds(qbase, _G)],
                                     osems[p])

    def compute(c, p, first):
        pltpu.make_async_copy(f_ref.at[idx_all.at[pl.ds(c * (_G * _K), _G * _K)]],
                              rows[p], gsems[p]).wait()

        @pl.when(jnp.logical_not(first))
        def _():
            out_store(c - 2, p).wait()

        for qp in range(_G // 2):
            wvec = w_all[pl.ds(c * (_G * _K) + qp * 16, 16)]
            for h in range(2):
                qq = qp * 2 + h
                accs = [jnp.zeros((16,), jnp.float32) for _ in range(4)]
                for k in range(_K):
                    wb = wvec.at[jnp.full((16,), h * 8 + k, jnp.int32)].get(
                        mode='promise_in_bounds')
                    for j in range(4):
                        row = rows[p][qq * _K + k, pl.ds(j * 16, 16)]
                        accs[j] = accs[j] + wb * row
                for j in range(4):
                    outs[p][qq, pl.ds(j * 16, 16)] = accs[j]
        out_store(c, p).start()

    gather(0, 0)
    gather(1, 1)

    def pair(pp, _):
        c0 = pp * 2
        compute(c0, 0, c0 == 0)

        @pl.when(c0 + 2 < nchunk)
        def _():
            gather(c0 + 2, 0)

        compute(c0 + 1, 1, c0 == 0)

        @pl.when(c0 + 3 < nchunk)
        def _():
            gather(c0 + 3, 1)

        return ()

    lax.fori_loop(0, nchunk // 2, pair, ())
    out_store(nchunk - 2, 0).wait()
    out_store(nchunk - 1, 1).wait()


@jax.jit
def kernel(query_coords, sensor_coords, sensor_features, sigma):
    B, Nq, _ = query_coords.shape
    Ns, F = sensor_features.shape[1], sensor_features.shape[2]
    nq_blocks = Nq // _TQ

    q_flat = query_coords.reshape(B * Nq, 3)
    s_t = sensor_coords.T
    sigma_safe = jax.nn.softplus(sigma) + 0.01
    coef = (-1.0 / (2.0 * sigma_safe * sigma_safe)).reshape(1, 1)

    idx8, w8 = pl.pallas_call(
        _tc_body,
        grid_spec=pltpu.PrefetchScalarGridSpec(
            num_scalar_prefetch=0,
            grid=(B, nq_blocks),
            in_specs=[
                pl.BlockSpec(memory_space=pltpu.SMEM),
                pl.BlockSpec((_TQ, 3), lambda b, i: (b * nq_blocks + i, 0)),
                pl.BlockSpec((3, Ns), lambda b, i: (0, 0)),
            ],
            out_specs=[
                pl.BlockSpec((_TQ, _K), lambda b, i: (b * nq_blocks + i, 0)),
                pl.BlockSpec((_TQ, _K), lambda b, i: (b * nq_blocks + i, 0)),
            ],
        ),
        out_shape=[
            jax.ShapeDtypeStruct((B * Nq, _K), jnp.int32),
            jax.ShapeDtypeStruct((B * Nq, _K), jnp.float32),
        ],
    )(coef, q_flat, s_t)

    feats_flat = sensor_features.reshape(B * Ns, F)
    idx_flat = idx8.reshape(B * Nq * _K)
    w_flat = w8.reshape(B * Nq * _K)

    sc = functools.partial(
        pl.kernel,
        out_type=jax.ShapeDtypeStruct((B * Nq, F), jnp.float32),
        mesh=plsc.VectorSubcoreMesh(core_axis_name="c", subcore_axis_name="s"),
        compiler_params=pltpu.CompilerParams(use_tc_tiling_on_sc=False),
        scratch_types=[
            pltpu.VMEM((512 * _K,), jnp.int32),
            pltpu.VMEM((512 * _K,), jnp.float32),
            pltpu.VMEM((_G * _K, F), jnp.float32),
            pltpu.VMEM((_G * _K, F), jnp.float32),
            pltpu.VMEM((_G, F), jnp.float32),
            pltpu.VMEM((_G, F), jnp.float32),
            pltpu.SemaphoreType.DMA,
            pltpu.SemaphoreType.DMA,
            pltpu.SemaphoreType.DMA,
            pltpu.SemaphoreType.DMA,
        ],
    )(_sc_body)

    out = sc(feats_flat, idx_flat, w_flat)
    return out.reshape(B, Nq, F)
